# baseline (device time: 468662 ns/iter reference)
import jax
import jax.numpy as jnp
from jax import lax
from jax.experimental import pallas as pl
from jax.experimental.pallas import tpu as pltpu

D = 4096
HALF = D // 2
C = 256
G = HALF // C
EPS = 1e-6
_MESH = pl.DeviceIdType.MESH


def _body(p_ref, r_ref, g_ref, out_ref, zbuf_ref, xbuf_ref,
          a_ref, b_ref, c_ref, o_ref,
          zs, zr, xs, xr, in_sems, out_sems):
    mx = lax.axis_index("x")
    my = lax.axis_index("y")
    mz = lax.axis_index("z")
    zpartner = (mx, my, 1 - mz)
    xneighbor = (1 - mx, my, mz)

    bar = pltpu.get_barrier_semaphore()
    pl.semaphore_signal(bar, inc=1, device_id=zpartner, device_id_type=_MESH)
    pl.semaphore_signal(bar, inc=1, device_id=xneighbor, device_id_type=_MESH)
    pl.semaphore_wait(bar, 2)

    zb = HALF * mx
    xb = HALF * (1 - mx)

    rdma_z = []
    for c in range(G):
        rz = pltpu.make_async_remote_copy(
            src_ref=p_ref.at[pl.ds(zb + c * C, C)],
            dst_ref=zbuf_ref.at[pl.ds(c * C, C)],
            send_sem=zs.at[c],
            recv_sem=zr.at[c],
            device_id=zpartner,
            device_id_type=_MESH,
        )
        rz.start()
        rdma_z.append(rz)

    rdma_x = [None] * G

    jobs = [("z", 0)]
    for c in range(1, G):
        jobs += [("z", c), ("x", c - 1)]
    jobs.append(("x", G - 1))

    def wait_arrival(kind, c):
        if kind == "z":
            rdma_z[c].wait_recv()
            rx = pltpu.make_async_remote_copy(
                src_ref=zbuf_ref.at[pl.ds(c * C, C)],
                dst_ref=xbuf_ref.at[pl.ds(c * C, C)],
                send_sem=xs.at[c],
                recv_sem=xr.at[c],
                device_id=xneighbor,
                device_id_type=_MESH,
            )
            rx.start()
            rdma_x[c] = rx
        else:
            rdma_x[c].wait_recv()

    def start_in(j):
        kind, c = jobs[j]
        s = j % 2
        base = (zb if kind == "z" else xb) + c * C
        buf = zbuf_ref if kind == "z" else xbuf_ref
        cps = [
            pltpu.make_async_copy(
                p_ref.at[pl.ds(base, C)], a_ref.at[s], in_sems.at[s, 0]),
            pltpu.make_async_copy(
                buf.at[pl.ds(c * C, C)], b_ref.at[s], in_sems.at[s, 1]),
            pltpu.make_async_copy(
                r_ref.at[pl.ds(base, C)], c_ref.at[s], in_sems.at[s, 2]),
        ]
        for cp in cps:
            cp.start()
        return cps

    ones = jnp.ones((D, 1), jnp.float32)
    pending_in = {}
    pending_out = {}
    wait_arrival(*jobs[0])
    pending_in[0] = start_in(0)
    for j in range(len(jobs)):
        if j + 1 < len(jobs):
            wait_arrival(*jobs[j + 1])
            pending_in[j + 1] = start_in(j + 1)
        for cp in pending_in.pop(j):
            cp.wait()
        s = j % 2
        if s in pending_out:
            pending_out.pop(s).wait()
        kind, c = jobs[j]
        y = a_ref[s] + b_ref[s] + c_ref[s]
        ss = lax.dot_general(
            y * y, ones,
            dimension_numbers=(((1,), (0,)), ((), ())),
            preferred_element_type=jnp.float32,
        )
        o_ref[s] = y * lax.rsqrt(ss * (1.0 / D) + EPS) * g_ref[...]
        base = (zb if kind == "z" else xb) + c * C
        cpo = pltpu.make_async_copy(
            o_ref.at[s], out_ref.at[pl.ds(base, C)], out_sems.at[s])
        cpo.start()
        pending_out[s] = cpo

    for cpo in pending_out.values():
        cpo.wait()
    for c in range(G):
        rdma_z[c].wait_send()
        rdma_x[c].wait_send()


def kernel(partial, resid, gamma):
    p = partial.reshape(D, D)
    g = gamma.reshape(1, D)
    out, _, _ = pl.pallas_call(
        _body,
        out_shape=[
            jax.ShapeDtypeStruct((D, D), jnp.float32),
            jax.ShapeDtypeStruct((HALF, D), jnp.float32),
            jax.ShapeDtypeStruct((HALF, D), jnp.float32),
        ],
        in_specs=[
            pl.BlockSpec(memory_space=pl.ANY),
            pl.BlockSpec(memory_space=pl.ANY),
            pl.BlockSpec(memory_space=pltpu.MemorySpace.VMEM),
        ],
        out_specs=[pl.BlockSpec(memory_space=pl.ANY)] * 3,
        scratch_shapes=[
            pltpu.VMEM((2, C, D), jnp.float32),
            pltpu.VMEM((2, C, D), jnp.float32),
            pltpu.VMEM((2, C, D), jnp.float32),
            pltpu.VMEM((2, C, D), jnp.float32),
            pltpu.SemaphoreType.DMA((G,)),
            pltpu.SemaphoreType.DMA((G,)),
            pltpu.SemaphoreType.DMA((G,)),
            pltpu.SemaphoreType.DMA((G,)),
            pltpu.SemaphoreType.DMA((2, 3)),
            pltpu.SemaphoreType.DMA((2,)),
        ],
        compiler_params=pltpu.CompilerParams(
            collective_id=0,
            vmem_limit_bytes=56 * 1024 * 1024,
        ),
    )(p, resid, g)
    return out


# device time: 441524 ns/iter; 1.0615x vs baseline; 1.0615x over previous
import jax
import jax.numpy as jnp
from jax import lax
from jax.experimental import pallas as pl
from jax.experimental.pallas import tpu as pltpu

D = 4096
HALF = D // 2
C = 128
G = HALF // C
EPS = 1e-6
_MESH = pl.DeviceIdType.MESH


def _body(p_ref, r_ref, g_ref, out_ref, zbuf_ref, xbuf_ref,
          a_ref, b_ref, c_ref, o_ref,
          zs, zr, xs, xr, in_sems, out_sems):
    mx = lax.axis_index("x")
    my = lax.axis_index("y")
    mz = lax.axis_index("z")
    zpartner = (mx, my, 1 - mz)
    xneighbor = (1 - mx, my, mz)

    bar = pltpu.get_barrier_semaphore()
    pl.semaphore_signal(bar, inc=1, device_id=zpartner, device_id_type=_MESH)
    pl.semaphore_signal(bar, inc=1, device_id=xneighbor, device_id_type=_MESH)
    pl.semaphore_wait(bar, 2)

    zb = HALF * mx
    xb = HALF * (1 - mx)

    rdma_z = []
    for c in range(G):
        rz = pltpu.make_async_remote_copy(
            src_ref=p_ref.at[pl.ds(zb + c * C, C)],
            dst_ref=zbuf_ref.at[pl.ds(c * C, C)],
            send_sem=zs.at[c],
            recv_sem=zr.at[c],
            device_id=zpartner,
            device_id_type=_MESH,
        )
        rz.start()
        rdma_z.append(rz)

    rdma_x = [None] * G

    jobs = [("z", 0)]
    for c in range(1, G):
        jobs += [("z", c), ("x", c - 1)]
    jobs.append(("x", G - 1))

    def wait_arrival(kind, c):
        if kind == "z":
            rdma_z[c].wait_recv()
            rx = pltpu.make_async_remote_copy(
                src_ref=zbuf_ref.at[pl.ds(c * C, C)],
                dst_ref=xbuf_ref.at[pl.ds(c * C, C)],
                send_sem=xs.at[c],
                recv_sem=xr.at[c],
                device_id=xneighbor,
                device_id_type=_MESH,
            )
            rx.start()
            rdma_x[c] = rx
        else:
            rdma_x[c].wait_recv()

    def start_in(j):
        kind, c = jobs[j]
        s = j % 2
        base = (zb if kind == "z" else xb) + c * C
        buf = zbuf_ref if kind == "z" else xbuf_ref
        cps = [
            pltpu.make_async_copy(
                p_ref.at[pl.ds(base, C)], a_ref.at[s], in_sems.at[s, 0]),
            pltpu.make_async_copy(
                buf.at[pl.ds(c * C, C)], b_ref.at[s], in_sems.at[s, 1]),
            pltpu.make_async_copy(
                r_ref.at[pl.ds(base, C)], c_ref.at[s], in_sems.at[s, 2]),
        ]
        for cp in cps:
            cp.start()
        return cps

    ones = jnp.ones((D, 1), jnp.float32)
    pending_in = {}
    pending_out = {}
    wait_arrival(*jobs[0])
    pending_in[0] = start_in(0)
    for j in range(len(jobs)):
        if j + 1 < len(jobs):
            wait_arrival(*jobs[j + 1])
            pending_in[j + 1] = start_in(j + 1)
        for cp in pending_in.pop(j):
            cp.wait()
        s = j % 2
        if s in pending_out:
            pending_out.pop(s).wait()
        kind, c = jobs[j]
        o_ref[s] = a_ref[s] + b_ref[s] + c_ref[s]
        base = (zb if kind == "z" else xb) + c * C
        cpo = pltpu.make_async_copy(
            o_ref.at[s], out_ref.at[pl.ds(base, C)], out_sems.at[s])
        cpo.start()
        pending_out[s] = cpo

    for cpo in pending_out.values():
        cpo.wait()
    for c in range(G):
        rdma_z[c].wait_send()
        rdma_x[c].wait_send()


def kernel(partial, resid, gamma):
    p = partial.reshape(D, D)
    g = gamma.reshape(1, D)
    out, _, _ = pl.pallas_call(
        _body,
        out_shape=[
            jax.ShapeDtypeStruct((D, D), jnp.float32),
            jax.ShapeDtypeStruct((HALF, D), jnp.float32),
            jax.ShapeDtypeStruct((HALF, D), jnp.float32),
        ],
        in_specs=[
            pl.BlockSpec(memory_space=pl.ANY),
            pl.BlockSpec(memory_space=pl.ANY),
            pl.BlockSpec(memory_space=pltpu.MemorySpace.VMEM),
        ],
        out_specs=[pl.BlockSpec(memory_space=pl.ANY)] * 3,
        scratch_shapes=[
            pltpu.VMEM((2, C, D), jnp.float32),
            pltpu.VMEM((2, C, D), jnp.float32),
            pltpu.VMEM((2, C, D), jnp.float32),
            pltpu.VMEM((2, C, D), jnp.float32),
            pltpu.SemaphoreType.DMA((G,)),
            pltpu.SemaphoreType.DMA((G,)),
            pltpu.SemaphoreType.DMA((G,)),
            pltpu.SemaphoreType.DMA((G,)),
            pltpu.SemaphoreType.DMA((2, 3)),
            pltpu.SemaphoreType.DMA((2,)),
        ],
        compiler_params=pltpu.CompilerParams(
            collective_id=0,
            vmem_limit_bytes=56 * 1024 * 1024,
        ),
    )(p, resid, g)
    return out


# device time: 355842 ns/iter; 1.3171x vs baseline; 1.2408x over previous
import jax
import jax.numpy as jnp
from jax import lax
from jax.experimental import pallas as pl
from jax.experimental.pallas import tpu as pltpu

D = 4096
Q = D // 4
C = 128
NQ = Q // C
EPS = 1e-6
_MESH = pl.DeviceIdType.MESH


def _body(p_ref, r_ref, g_ref, out_ref, zland, lland, rland,
          a_ref, b_ref, c_ref, o_ref,
          zs, zr, rss, lrr, lss, rrr, in_sems, out_sems):
    mx = lax.axis_index("x")
    my = lax.axis_index("y")
    mz = lax.axis_index("z")
    d = (1 - mx) * my + mx * (3 - my)
    dl = (d + 3) % 4
    dr = (d + 1) % 4
    dop = (d + 2) % 4
    zpart = (mx, my, 1 - mz)
    right = (my, 1 - mx, mz)
    left = (1 - my, mx, mz)

    bar = pltpu.get_barrier_semaphore()
    for peer in (zpart, right, left):
        pl.semaphore_signal(bar, inc=1, device_id=peer, device_id_type=_MESH)
    pl.semaphore_wait(bar, 3)

    q0 = Q * d

    zorder = [0, 4, 1, 5, 2, 6, 3, 7]
    rz = {}
    for k in zorder:
        rdma = pltpu.make_async_remote_copy(
            src_ref=p_ref.at[pl.ds(q0 + C * k, C)],
            dst_ref=zland.at[pl.ds(C * k, C)],
            send_sem=zs.at[k],
            recv_sem=zr.at[k],
            device_id=zpart,
            device_id_type=_MESH,
        )
        rdma.start()
        rz[k] = rdma

    rR = {}
    rL = {}

    def issue_R(slot, src_buf, src_row):
        rdma = pltpu.make_async_remote_copy(
            src_ref=src_buf.at[pl.ds(src_row, C)],
            dst_ref=lland.at[pl.ds(C * slot, C)],
            send_sem=rss.at[slot],
            recv_sem=lrr.at[slot],
            device_id=right,
            device_id_type=_MESH,
        )
        rdma.start()
        rR[slot] = rdma

    def issue_L(slot, src_buf, src_row):
        rdma = pltpu.make_async_remote_copy(
            src_ref=src_buf.at[pl.ds(src_row, C)],
            dst_ref=rland.at[pl.ds(C * slot, C)],
            send_sem=lss.at[slot],
            recv_sem=rrr.at[slot],
            device_id=left,
            device_id_type=_MESH,
        )
        rdma.start()
        rL[slot] = rdma

    def z_pre(k):
        def f():
            rz[k].wait_recv()
            if k < 4:
                issue_R(k, zland, C * k)
                issue_L(8 + k, zland, C * k)
            else:
                kb = k - 4
                issue_L(kb, zland, C * k)
                issue_R(8 + kb, zland, C * k)
        return f

    def lr_pre(s):
        def f():
            rR[s].wait_recv()
            if s < 4:
                issue_R(4 + s, lland, C * s)
        return f

    def rr_pre(s):
        def f():
            rL[s].wait_recv()
            if s < 4:
                issue_L(4 + s, rland, C * s)
        return f

    jobs = []
    for i in range(4):
        jobs.append((z_pre(i), zland, C * i, q0 + C * i))
        jobs.append((z_pre(4 + i), zland, C * (4 + i), q0 + C * (4 + i)))
        if i >= 1:
            s = i - 1
            jobs.append((lr_pre(s), lland, C * s, Q * dl + C * s))
            jobs.append((rr_pre(s), rland, C * s, Q * dr + Q // 2 + C * s))
    jobs.append((lr_pre(3), lland, C * 3, Q * dl + C * 3))
    jobs.append((rr_pre(3), rland, C * 3, Q * dr + Q // 2 + C * 3))
    for i in range(4):
        jobs.append((lr_pre(8 + i), lland, C * (8 + i),
                     Q * dl + Q // 2 + C * i))
        jobs.append((rr_pre(8 + i), rland, C * (8 + i), Q * dr + C * i))
        jobs.append((lr_pre(4 + i), lland, C * (4 + i), Q * dop + C * i))
        jobs.append((rr_pre(4 + i), rland, C * (4 + i),
                     Q * dop + Q // 2 + C * i))
    assert len(jobs) == 32

    def start_in(j):
        _, buf, brow, orow = jobs[j]
        s = j % 2
        cps = [
            pltpu.make_async_copy(
                p_ref.at[pl.ds(orow, C)], a_ref.at[s], in_sems.at[s, 0]),
            pltpu.make_async_copy(
                buf.at[pl.ds(brow, C)], b_ref.at[s], in_sems.at[s, 1]),
            pltpu.make_async_copy(
                r_ref.at[pl.ds(orow, C)], c_ref.at[s], in_sems.at[s, 2]),
        ]
        for cp in cps:
            cp.start()
        return cps

    pending_in = {}
    pending_out = {}
    jobs[0][0]()
    pending_in[0] = start_in(0)
    for j in range(len(jobs)):
        if j + 1 < len(jobs):
            jobs[j + 1][0]()
            pending_in[j + 1] = start_in(j + 1)
        for cp in pending_in.pop(j):
            cp.wait()
        s = j % 2
        if s in pending_out:
            pending_out.pop(s).wait()
        y = a_ref[s] + b_ref[s] + c_ref[s]
        ms = jnp.mean(y * y, axis=-1, keepdims=True)
        o_ref[s] = y * lax.rsqrt(ms + EPS) * g_ref[...]
        orow = jobs[j][3]
        cpo = pltpu.make_async_copy(
            o_ref.at[s], out_ref.at[pl.ds(orow, C)], out_sems.at[s])
        cpo.start()
        pending_out[s] = cpo

    for cpo in pending_out.values():
        cpo.wait()
    for rdma in list(rz.values()) + list(rR.values()) + list(rL.values()):
        rdma.wait_send()


def kernel(partial, resid, gamma):
    p = partial.reshape(D, D)
    g = gamma.reshape(1, D)
    out, _, _, _ = pl.pallas_call(
        _body,
        out_shape=[
            jax.ShapeDtypeStruct((D, D), jnp.float32),
            jax.ShapeDtypeStruct((Q, D), jnp.float32),
            jax.ShapeDtypeStruct((12 * C, D), jnp.float32),
            jax.ShapeDtypeStruct((12 * C, D), jnp.float32),
        ],
        in_specs=[
            pl.BlockSpec(memory_space=pl.ANY),
            pl.BlockSpec(memory_space=pl.ANY),
            pl.BlockSpec(memory_space=pltpu.MemorySpace.VMEM),
        ],
        out_specs=[pl.BlockSpec(memory_space=pl.ANY)] * 4,
        scratch_shapes=[
            pltpu.VMEM((2, C, D), jnp.float32),
            pltpu.VMEM((2, C, D), jnp.float32),
            pltpu.VMEM((2, C, D), jnp.float32),
            pltpu.VMEM((2, C, D), jnp.float32),
            pltpu.SemaphoreType.DMA((NQ,)),
            pltpu.SemaphoreType.DMA((NQ,)),
            pltpu.SemaphoreType.DMA((12,)),
            pltpu.SemaphoreType.DMA((12,)),
            pltpu.SemaphoreType.DMA((12,)),
            pltpu.SemaphoreType.DMA((12,)),
            pltpu.SemaphoreType.DMA((2, 3)),
            pltpu.SemaphoreType.DMA((2,)),
        ],
        compiler_params=pltpu.CompilerParams(
            collective_id=0,
            vmem_limit_bytes=56 * 1024 * 1024,
        ),
    )(p, resid, g)
    return out


# device time: 339967 ns/iter; 1.3786x vs baseline; 1.0467x over previous
import jax
import jax.numpy as jnp
from jax import lax
from jax.experimental import pallas as pl
from jax.experimental.pallas import tpu as pltpu

D = 4096
C = 128
NZ = 10
NP = 11
EPS = 1e-6
_MESH = pl.DeviceIdType.MESH


def _body(p_ref, r_ref, g_ref, out_ref, zland, lland, rland,
          a_ref, b_ref, c_ref, o_ref,
          zs, zr, rss, lrr, lss, rrr, in_sems, out_sems):
    mx = lax.axis_index("x")
    my = lax.axis_index("y")
    mz = lax.axis_index("z")
    d = (1 - mx) * my + mx * (3 - my)
    dl = (d + 3) % 4
    dr = (d + 1) % 4
    dop = (d + 2) % 4
    my_p = 24 + 4 * (d % 2)
    other_p = 24 + 4 * ((d + 1) % 2)
    zpart = (mx, my, 1 - mz)
    right = (my, 1 - mx, mz)
    left = (1 - my, mx, mz)

    bar = pltpu.get_barrier_semaphore()
    for peer in (zpart, right, left):
        pl.semaphore_signal(bar, inc=1, device_id=peer, device_id_type=_MESH)
    pl.semaphore_wait(bar, 3)

    def z_chunk_row(k):
        if k < 6:
            return C * (6 * d + k)
        return C * (my_p + (k - 6))

    zorder = [0, 3, 1, 4, 2, 5, 6, 8, 7, 9]
    rz = {}
    for k in zorder:
        rdma = pltpu.make_async_remote_copy(
            src_ref=p_ref.at[pl.ds(z_chunk_row(k), C)],
            dst_ref=zland.at[pl.ds(C * k, C)],
            send_sem=zs.at[k],
            recv_sem=zr.at[k],
            device_id=zpart,
            device_id_type=_MESH,
        )
        rdma.start()
        rz[k] = rdma

    rR = {}
    rL = {}

    def issue_R(slot, src_buf, src_row):
        rdma = pltpu.make_async_remote_copy(
            src_ref=src_buf.at[pl.ds(src_row, C)],
            dst_ref=lland.at[pl.ds(C * slot, C)],
            send_sem=rss.at[slot],
            recv_sem=lrr.at[slot],
            device_id=right,
            device_id_type=_MESH,
        )
        rdma.start()
        rR[slot] = rdma

    def issue_L(slot, src_buf, src_row):
        rdma = pltpu.make_async_remote_copy(
            src_ref=src_buf.at[pl.ds(src_row, C)],
            dst_ref=rland.at[pl.ds(C * slot, C)],
            send_sem=lss.at[slot],
            recv_sem=rrr.at[slot],
            device_id=left,
            device_id_type=_MESH,
        )
        rdma.start()
        rL[slot] = rdma

    def z_pre(k):
        def f():
            rz[k].wait_recv()
            if k < 3:
                issue_R(k, zland, C * k)
                issue_L(6 + k, zland, C * k)
            elif k < 6:
                kb = k - 3
                issue_L(kb, zland, C * k)
                issue_R(6 + kb, zland, C * k)
            elif k < 8:
                issue_R(9 + (k - 6), zland, C * k)
            else:
                issue_L(9 + (k - 8), zland, C * k)
        return f

    def lr_pre(s):
        def f():
            rR[s].wait_recv()
            if s < 3:
                issue_R(3 + s, lland, C * s)
        return f

    def rr_pre(s):
        def f():
            rL[s].wait_recv()
            if s < 3:
                issue_L(3 + s, rland, C * s)
        return f

    def lr_row(s):
        if s < 3:
            return C * (6 * dl + s)
        if s < 6:
            return C * (6 * dop + (s - 3))
        if s < 9:
            return C * (6 * dl + 3 + (s - 6))
        return C * (other_p + (s - 9))

    def rr_row(s):
        if s < 3:
            return C * (6 * dr + 3 + s)
        if s < 6:
            return C * (6 * dop + 3 + (s - 3))
        if s < 9:
            return C * (6 * dr + (s - 6))
        return C * (other_p + 2 + (s - 9))

    jobs = []

    def zjob(k):
        jobs.append((z_pre(k), zland, C * k, z_chunk_row(k)))

    def lrjob(s):
        jobs.append((lr_pre(s), lland, C * s, lr_row(s)))

    def rrjob(s):
        jobs.append((rr_pre(s), rland, C * s, rr_row(s)))

    zjob(0); zjob(3)
    lrjob(0); rrjob(0)
    zjob(1)
    lrjob(6); rrjob(6)
    zjob(4)
    lrjob(1); rrjob(1)
    lrjob(3); rrjob(3)
    zjob(2)
    lrjob(7); rrjob(7)
    zjob(5)
    lrjob(2); rrjob(2)
    lrjob(4); rrjob(4)
    zjob(6); zjob(8)
    lrjob(8); rrjob(8)
    zjob(7); zjob(9)
    lrjob(9); rrjob(9)
    lrjob(5); rrjob(5)
    lrjob(10); rrjob(10)
    assert len(jobs) == 32

    def start_in(j):
        _, buf, brow, orow = jobs[j]
        s = j % 2
        cps = [
            pltpu.make_async_copy(
                p_ref.at[pl.ds(orow, C)], a_ref.at[s], in_sems.at[s, 0]),
            pltpu.make_async_copy(
                buf.at[pl.ds(brow, C)], b_ref.at[s], in_sems.at[s, 1]),
            pltpu.make_async_copy(
                r_ref.at[pl.ds(orow, C)], c_ref.at[s], in_sems.at[s, 2]),
        ]
        for cp in cps:
            cp.start()
        return cps

    pending_in = {}
    pending_out = {}
    jobs[0][0]()
    pending_in[0] = start_in(0)
    for j in range(len(jobs)):
        if j + 1 < len(jobs):
            jobs[j + 1][0]()
            pending_in[j + 1] = start_in(j + 1)
        for cp in pending_in.pop(j):
            cp.wait()
        s = j % 2
        if s in pending_out:
            pending_out.pop(s).wait()
        y = a_ref[s] + b_ref[s] + c_ref[s]
        ms = jnp.mean(y * y, axis=-1, keepdims=True)
        o_ref[s] = y * lax.rsqrt(ms + EPS) * g_ref[...]
        orow = jobs[j][3]
        cpo = pltpu.make_async_copy(
            o_ref.at[s], out_ref.at[pl.ds(orow, C)], out_sems.at[s])
        cpo.start()
        pending_out[s] = cpo

    for cpo in pending_out.values():
        cpo.wait()
    for rdma in list(rz.values()) + list(rR.values()) + list(rL.values()):
        rdma.wait_send()


def kernel(partial, resid, gamma):
    p = partial.reshape(D, D)
    g = gamma.reshape(1, D)
    out, _, _, _ = pl.pallas_call(
        _body,
        out_shape=[
            jax.ShapeDtypeStruct((D, D), jnp.float32),
            jax.ShapeDtypeStruct((NZ * C, D), jnp.float32),
            jax.ShapeDtypeStruct((NP * C, D), jnp.float32),
            jax.ShapeDtypeStruct((NP * C, D), jnp.float32),
        ],
        in_specs=[
            pl.BlockSpec(memory_space=pl.ANY),
            pl.BlockSpec(memory_space=pl.ANY),
            pl.BlockSpec(memory_space=pltpu.MemorySpace.VMEM),
        ],
        out_specs=[pl.BlockSpec(memory_space=pl.ANY)] * 4,
        scratch_shapes=[
            pltpu.VMEM((2, C, D), jnp.float32),
            pltpu.VMEM((2, C, D), jnp.float32),
            pltpu.VMEM((2, C, D), jnp.float32),
            pltpu.VMEM((2, C, D), jnp.float32),
            pltpu.SemaphoreType.DMA((NZ,)),
            pltpu.SemaphoreType.DMA((NZ,)),
            pltpu.SemaphoreType.DMA((NP,)),
            pltpu.SemaphoreType.DMA((NP,)),
            pltpu.SemaphoreType.DMA((NP,)),
            pltpu.SemaphoreType.DMA((NP,)),
            pltpu.SemaphoreType.DMA((2, 3)),
            pltpu.SemaphoreType.DMA((2,)),
        ],
        compiler_params=pltpu.CompilerParams(
            collective_id=0,
            vmem_limit_bytes=56 * 1024 * 1024,
        ),
    )(p, resid, g)
    return out
